# baseline (speedup 1.0000x reference)
import jax
import jax.numpy as jnp
from jax.experimental import pallas as pl
from jax.experimental.pallas import tpu as pltpu


def _body(inst_ref, feat_ref, w1_ref, w2_ref, w3_ref, b3_ref, out_ref, acc_ref):
    z = pl.program_id(1)

    @pl.when(z == 0)
    def _init():
        acc_ref[...] = jnp.zeros_like(acc_ref)

    x = inst_ref[0, 0]
    f = feat_ref[0]

    rows = []
    for idv in range(1, 16):
        cnt8 = jnp.sum((x == idv).astype(jnp.float32), axis=0)
        win = (cnt8 >= 3.5).astype(jnp.float32)
        rows.append(win.reshape(1, -1))
    m = jnp.concatenate(rows, axis=0)

    s = jax.lax.dot_general(m, f, (((1,), (1,)), ((), ())),
                            preferred_element_type=jnp.float32)
    cnt = jnp.sum(m, axis=1).reshape(15, 1)
    acc_ref[1:16, 0:64] += s
    acc_ref[1:16, 64:65] += cnt

    @pl.when(z == 31)
    def _finish():
        a = acc_ref[...]
        sums = a[:, 0:64]
        cv = a[:, 64:65]
        emb = sums / jnp.where(cv > 0, cv, 1.0)
        h = jax.lax.dot_general(emb, w1_ref[...], (((1,), (1,)), ((), ())),
                                preferred_element_type=jnp.float32)
        h = jnp.maximum(h, 0.0)
        h = jax.lax.dot_general(h, w2_ref[...], (((1,), (1,)), ((), ())),
                                preferred_element_type=jnp.float32)
        h = jnp.maximum(h, 0.0)
        y = jax.lax.dot_general(h, w3_ref[...], (((1,), (1,)), ((), ())),
                                preferred_element_type=jnp.float32) + b3_ref[...]
        y = jnp.where(cv > 0, y, jnp.nan)
        out_ref[0] = y


def kernel(features, instances, W1, W2, W3, b3):
    B, C, Z, H, W = features.shape
    inst8 = (instances.reshape(B, Z, 2, H, 2, W, 2)
             .transpose(0, 1, 2, 4, 6, 3, 5)
             .reshape(B, Z, 8, H * W))
    feats2 = features.reshape(B, C, Z * H * W)

    outp = pl.pallas_call(
        _body,
        grid=(B, Z),
        in_specs=[
            pl.BlockSpec((1, 1, 8, H * W), lambda b, z: (b, z, 0, 0)),
            pl.BlockSpec((1, C, H * W), lambda b, z: (b, 0, z)),
            pl.BlockSpec((64, 64), lambda b, z: (0, 0)),
            pl.BlockSpec((64, 64), lambda b, z: (0, 0)),
            pl.BlockSpec((32, 64), lambda b, z: (0, 0)),
            pl.BlockSpec((1, 32), lambda b, z: (0, 0)),
        ],
        out_specs=pl.BlockSpec((1, 16, 32), lambda b, z: (b, 0, 0)),
        out_shape=jax.ShapeDtypeStruct((B, 16, 32), jnp.float32),
        scratch_shapes=[pltpu.VMEM((16, 128), jnp.float32)],
    )(inst8, feats2, W1, W2, W3, b3.reshape(1, 32))

    return outp[:, 1:, :].reshape(B * 15, 32)


# speedup vs baseline: 4.5419x; 4.5419x over previous
import jax
import jax.numpy as jnp
from jax import lax
from jax.experimental import pallas as pl
from jax.experimental.pallas import tpu as pltpu


def _body(inst_ref, feat_ref, w1_ref, w2_ref, w3_ref, b3_ref, out_ref, acc_ref):
    z = pl.program_id(1)

    @pl.when(z == 0)
    def _init():
        acc_ref[...] = jnp.zeros_like(acc_ref)

    x = inst_ref[0]
    f = feat_ref[0]

    rows = []
    for idv in range(1, 16):
        e = (x == idv).astype(jnp.bfloat16)
        zs = e[0] + e[1]
        rows.append(zs.reshape(1, 128, 128))
    zsum = jnp.concatenate(rows, axis=0)

    r = lax.broadcasted_iota(jnp.int32, (128, 64), 0)
    c = lax.broadcasted_iota(jnp.int32, (128, 64), 1)
    pair = ((r // 2) == c).astype(jnp.bfloat16)

    t = lax.dot_general(zsum, pair, (((1,), (0,)), ((), ())),
                        preferred_element_type=jnp.float32)
    u = lax.dot_general(t.astype(jnp.bfloat16), pair, (((1,), (0,)), ((), ())),
                        preferred_element_type=jnp.float32)
    m = (u >= 3.5).astype(jnp.float32).reshape(15, 4096)

    s = lax.dot_general(m, f, (((1,), (1,)), ((), ())),
                        preferred_element_type=jnp.float32)
    cnt = jnp.sum(m, axis=1).reshape(15, 1)
    acc_ref[1:16, 0:64] += s
    acc_ref[1:16, 64:65] += cnt

    @pl.when(z == 31)
    def _finish():
        a = acc_ref[...]
        sums = a[:, 0:64]
        cv = a[:, 64:65]
        emb = sums / jnp.where(cv > 0, cv, 1.0)
        h = lax.dot_general(emb, w1_ref[...], (((1,), (1,)), ((), ())),
                            preferred_element_type=jnp.float32)
        h = jnp.maximum(h, 0.0)
        h = lax.dot_general(h, w2_ref[...], (((1,), (1,)), ((), ())),
                            preferred_element_type=jnp.float32)
        h = jnp.maximum(h, 0.0)
        y = lax.dot_general(h, w3_ref[...], (((1,), (1,)), ((), ())),
                            preferred_element_type=jnp.float32) + b3_ref[...]
        y = jnp.where(cv > 0, y, jnp.nan)
        out_ref[0] = y


def kernel(features, instances, W1, W2, W3, b3):
    B, C, Z, H, W = features.shape
    feats2 = features.reshape(B, C, Z * H * W)

    outp = pl.pallas_call(
        _body,
        grid=(B, Z),
        in_specs=[
            pl.BlockSpec((1, 2, 2 * H, 2 * W), lambda b, z: (b, z, 0, 0)),
            pl.BlockSpec((1, C, H * W), lambda b, z: (b, 0, z)),
            pl.BlockSpec((64, 64), lambda b, z: (0, 0)),
            pl.BlockSpec((64, 64), lambda b, z: (0, 0)),
            pl.BlockSpec((32, 64), lambda b, z: (0, 0)),
            pl.BlockSpec((1, 32), lambda b, z: (0, 0)),
        ],
        out_specs=pl.BlockSpec((1, 16, 32), lambda b, z: (b, 0, 0)),
        out_shape=jax.ShapeDtypeStruct((B, 16, 32), jnp.float32),
        scratch_shapes=[pltpu.VMEM((16, 128), jnp.float32)],
    )(instances, feats2, W1, W2, W3, b3.reshape(1, 32))

    return outp[:, 1:, :].reshape(B * 15, 32)


# speedup vs baseline: 5.1103x; 1.1252x over previous
import jax
import jax.numpy as jnp
from jax import lax
from jax.experimental import pallas as pl
from jax.experimental.pallas import tpu as pltpu

_ZB = 2


def _winner_masks(x):
    rows = []
    for idv in range(1, 16):
        e = (x == idv).astype(jnp.bfloat16)
        zs = e[0] + e[1]
        rows.append(zs.reshape(1, 128, 128))
    zsum = jnp.concatenate(rows, axis=0)

    r = lax.broadcasted_iota(jnp.int32, (128, 64), 0)
    c = lax.broadcasted_iota(jnp.int32, (128, 64), 1)
    pair = ((r // 2) == c).astype(jnp.bfloat16)

    t = lax.dot_general(zsum, pair, (((1,), (0,)), ((), ())),
                        preferred_element_type=jnp.float32)
    u = lax.dot_general(t.astype(jnp.bfloat16), pair, (((1,), (0,)), ((), ())),
                        preferred_element_type=jnp.float32)
    return (u >= 3.5).astype(jnp.float32).reshape(15, 4096)


def _body(inst_ref, feat_ref, w1_ref, w2_ref, w3_ref, b3_ref, out_ref, acc_ref):
    zz = pl.program_id(1)
    nzz = pl.num_programs(1)

    @pl.when(zz == 0)
    def _init():
        acc_ref[...] = jnp.zeros_like(acc_ref)

    x = inst_ref[0]
    f = feat_ref[0]

    s = jnp.zeros((15, 64), jnp.float32)
    cnt = jnp.zeros((15, 1), jnp.float32)
    for dz in range(_ZB):
        m = _winner_masks(x[2 * dz:2 * dz + 2])
        fz = f[:, dz * 4096:(dz + 1) * 4096]
        s += lax.dot_general(m, fz, (((1,), (1,)), ((), ())),
                             preferred_element_type=jnp.float32)
        cnt += jnp.sum(m, axis=1).reshape(15, 1)
    acc_ref[1:16, 0:64] += s
    acc_ref[1:16, 64:65] += cnt

    @pl.when(zz == nzz - 1)
    def _finish():
        a = acc_ref[...]
        sums = a[:, 0:64]
        cv = a[:, 64:65]
        emb = sums / jnp.where(cv > 0, cv, 1.0)
        h = lax.dot_general(emb, w1_ref[...], (((1,), (1,)), ((), ())),
                            preferred_element_type=jnp.float32)
        h = jnp.maximum(h, 0.0)
        h = lax.dot_general(h, w2_ref[...], (((1,), (1,)), ((), ())),
                            preferred_element_type=jnp.float32)
        h = jnp.maximum(h, 0.0)
        y = lax.dot_general(h, w3_ref[...], (((1,), (1,)), ((), ())),
                            preferred_element_type=jnp.float32) + b3_ref[...]
        y = jnp.where(cv > 0, y, jnp.nan)
        out_ref[0] = y


def kernel(features, instances, W1, W2, W3, b3):
    B, C, Z, H, W = features.shape
    feats2 = features.reshape(B, C, Z * H * W)

    outp = pl.pallas_call(
        _body,
        grid=(B, Z // _ZB),
        in_specs=[
            pl.BlockSpec((1, 2 * _ZB, 2 * H, 2 * W), lambda b, z: (b, z, 0, 0)),
            pl.BlockSpec((1, C, _ZB * H * W), lambda b, z: (b, 0, z)),
            pl.BlockSpec((64, 64), lambda b, z: (0, 0)),
            pl.BlockSpec((64, 64), lambda b, z: (0, 0)),
            pl.BlockSpec((32, 64), lambda b, z: (0, 0)),
            pl.BlockSpec((1, 32), lambda b, z: (0, 0)),
        ],
        out_specs=pl.BlockSpec((1, 16, 32), lambda b, z: (b, 0, 0)),
        out_shape=jax.ShapeDtypeStruct((B, 16, 32), jnp.float32),
        scratch_shapes=[pltpu.VMEM((16, 128), jnp.float32)],
    )(instances, feats2, W1, W2, W3, b3.reshape(1, 32))

    return outp[:, 1:, :].reshape(B * 15, 32)


# speedup vs baseline: 5.1515x; 1.0081x over previous
import jax
import jax.numpy as jnp
from jax import lax
from jax.experimental import pallas as pl
from jax.experimental.pallas import tpu as pltpu

_ZB = 4


def _winner_masks(x):
    rows = []
    for idv in range(1, 16):
        e = (x == idv).astype(jnp.bfloat16)
        zs = e[0] + e[1]
        rows.append(zs.reshape(1, 128, 128))
    zsum = jnp.concatenate(rows, axis=0)

    r = lax.broadcasted_iota(jnp.int32, (128, 64), 0)
    c = lax.broadcasted_iota(jnp.int32, (128, 64), 1)
    pair = ((r // 2) == c).astype(jnp.bfloat16)

    t = lax.dot_general(zsum, pair, (((1,), (0,)), ((), ())),
                        preferred_element_type=jnp.float32)
    u = lax.dot_general(t.astype(jnp.bfloat16), pair, (((1,), (0,)), ((), ())),
                        preferred_element_type=jnp.float32)
    return (u >= 3.5).astype(jnp.float32).reshape(15, 4096)


def _body(inst_ref, feat_ref, w1_ref, w2_ref, w3_ref, b3_ref, out_ref, acc_ref):
    zz = pl.program_id(1)
    nzz = pl.num_programs(1)

    @pl.when(zz == 0)
    def _init():
        acc_ref[...] = jnp.zeros_like(acc_ref)

    x = inst_ref[0]
    f = feat_ref[0]

    s = jnp.zeros((15, 64), jnp.float32)
    cnt = jnp.zeros((15, 1), jnp.float32)
    for dz in range(_ZB):
        m = _winner_masks(x[2 * dz:2 * dz + 2])
        fz = f[:, dz * 4096:(dz + 1) * 4096]
        s += lax.dot_general(m, fz, (((1,), (1,)), ((), ())),
                             preferred_element_type=jnp.float32)
        cnt += jnp.sum(m, axis=1).reshape(15, 1)
    acc_ref[1:16, 0:64] += s
    acc_ref[1:16, 64:65] += cnt

    @pl.when(zz == nzz - 1)
    def _finish():
        a = acc_ref[...]
        sums = a[:, 0:64]
        cv = a[:, 64:65]
        emb = sums / jnp.where(cv > 0, cv, 1.0)
        h = lax.dot_general(emb, w1_ref[...], (((1,), (1,)), ((), ())),
                            preferred_element_type=jnp.float32)
        h = jnp.maximum(h, 0.0)
        h = lax.dot_general(h, w2_ref[...], (((1,), (1,)), ((), ())),
                            preferred_element_type=jnp.float32)
        h = jnp.maximum(h, 0.0)
        y = lax.dot_general(h, w3_ref[...], (((1,), (1,)), ((), ())),
                            preferred_element_type=jnp.float32) + b3_ref[...]
        y = jnp.where(cv > 0, y, jnp.nan)
        out_ref[0] = y


def kernel(features, instances, W1, W2, W3, b3):
    B, C, Z, H, W = features.shape
    feats2 = features.reshape(B, C, Z * H * W)

    outp = pl.pallas_call(
        _body,
        grid=(B, Z // _ZB),
        in_specs=[
            pl.BlockSpec((1, 2 * _ZB, 2 * H, 2 * W), lambda b, z: (b, z, 0, 0)),
            pl.BlockSpec((1, C, _ZB * H * W), lambda b, z: (b, 0, z)),
            pl.BlockSpec((64, 64), lambda b, z: (0, 0)),
            pl.BlockSpec((64, 64), lambda b, z: (0, 0)),
            pl.BlockSpec((32, 64), lambda b, z: (0, 0)),
            pl.BlockSpec((1, 32), lambda b, z: (0, 0)),
        ],
        out_specs=pl.BlockSpec((1, 16, 32), lambda b, z: (b, 0, 0)),
        out_shape=jax.ShapeDtypeStruct((B, 16, 32), jnp.float32),
        scratch_shapes=[pltpu.VMEM((16, 128), jnp.float32)],
    )(instances, feats2, W1, W2, W3, b3.reshape(1, 32))

    return outp[:, 1:, :].reshape(B * 15, 32)
